# baseline (speedup 1.0000x reference)
import functools

import jax
import jax.numpy as jnp
from jax import lax
from jax.experimental import pallas as pl
from jax.experimental.pallas import tpu as pltpu
from jax.experimental.pallas import tpu_sc as plsc

N, D, E, R = 2048, 768, 64, 64
B = 32
SH = 5
GC = 32
PN = N + (R - 1) * B
PN = ((PN + GC * B - 1) // (GC * B)) * (GC * B)
T = PN // B
TP = T // GC
TBL = ((T + TP + 1 + 7) // 8) * 8

NC, NS = 2, 16
NW = NC * NS
ROWS_PER_W = N // NW

_SQRT_HALF = 0.7071067811865476


def _gelu_exact(v):
    return 0.5 * v * (1.0 + jax.lax.erf(v * _SQRT_HALF))


_RB = 1024
_RG = N // _RB


def _routing_body(rules_full_ref, rules_blk_ref, pos_ref, tbl_ref):
    pid = pl.program_id(0)
    rules_full = rules_full_ref[...]
    r_iota = lax.broadcasted_iota(jnp.int32, (R, N), 0)
    eqc = (r_iota == rules_full).astype(jnp.float32)
    counts_col = jnp.sum(eqc, axis=1, keepdims=True).astype(jnp.int32)
    padded_col = ((counts_col + (B - 1)) >> SH) << SH
    tri_r = (lax.broadcasted_iota(jnp.int32, (R, R), 1)
             < lax.broadcasted_iota(jnp.int32, (R, R), 0)).astype(jnp.float32)
    offsets_col = lax.dot_general(tri_r, padded_col.astype(jnp.float32),
                                  (((1,), (0,)), ((), ())),
                                  preferred_element_type=jnp.float32)

    @pl.when(pid == 0)
    def _chunks():
        nvalid = jnp.sum(padded_col) >> SH
        nvp = (nvalid + GC - 1) // GC
        p_iota = lax.broadcasted_iota(jnp.int32, (1, TP), 1)
        pair_tgt = jnp.minimum(p_iota, nvp)
        cb = (lax.broadcasted_iota(jnp.int32, (R, T), 1) * B).astype(jnp.float32)
        le = (offsets_col <= cb).astype(jnp.int32)
        rule = jnp.sum(le, axis=0, keepdims=True) - 1
        nv = jnp.full((1, TBL - T - TP), nvalid, jnp.int32)
        tbl_ref[...] = jnp.concatenate([pair_tgt, rule, nv], axis=1)

    rules_blk = rules_blk_ref[...]
    eqb = (lax.broadcasted_iota(jnp.int32, (R, _RB), 0)
           == rules_blk).astype(jnp.float32)
    before = (lax.broadcasted_iota(jnp.int32, (R, N), 1)
              < pid * _RB).astype(jnp.float32)
    pc_col = jnp.sum(eqc * before, axis=1, keepdims=True)
    tri_b = (lax.broadcasted_iota(jnp.int32, (_RB, _RB), 0)
             < lax.broadcasted_iota(jnp.int32, (_RB, _RB), 1)
             ).astype(jnp.float32)
    cblk = lax.dot_general(eqb, tri_b, (((1,), (0,)), ((), ())),
                           preferred_element_type=jnp.float32)
    rank = jnp.sum(eqb * (pc_col + cblk), axis=0, keepdims=True)
    off_tok = jnp.sum(eqb * offsets_col, axis=0, keepdims=True)
    pos_ref[...] = (off_tok + rank).astype(jnp.int32)


def _compute_routing(rules):
    rules_row = rules.reshape(1, N)
    pos, tbl = pl.pallas_call(
        _routing_body,
        grid=(_RG,),
        in_specs=[
            pl.BlockSpec((1, N), lambda i: (0, 0)),
            pl.BlockSpec((1, _RB), lambda i: (0, i)),
        ],
        out_specs=[
            pl.BlockSpec((1, _RB), lambda i: (0, i)),
            pl.BlockSpec((1, TBL), lambda i: (0, 0)),
        ],
        out_shape=[
            jax.ShapeDtypeStruct((1, N), jnp.int32),
            jax.ShapeDtypeStruct((1, TBL), jnp.int32),
        ],
    )(rules_row, rules_row)
    return pos.reshape(N), tbl.reshape(TBL)


@functools.lru_cache(maxsize=None)
def _sc_kernels():
    mesh = plsc.VectorSubcoreMesh(core_axis_name="c", subcore_axis_name="s")
    scratch = [
        pltpu.VMEM((ROWS_PER_W,), jnp.int32),
        pltpu.VMEM((ROWS_PER_W, D), jnp.float32),
        pltpu.SemaphoreType.DMA,
    ]

    @functools.partial(
        pl.kernel,
        mesh=mesh,
        out_type=jax.ShapeDtypeStruct((PN, D), jnp.float32),
        scratch_types=scratch,
    )
    def sc_scatter(x_hbm, pos_hbm, out_hbm, idx_v, rows_v, sem):
        wid = lax.axis_index("s") * NC + lax.axis_index("c")
        base = wid * ROWS_PER_W
        pltpu.sync_copy(pos_hbm.at[pl.ds(base, ROWS_PER_W)], idx_v)
        pltpu.sync_copy(x_hbm.at[pl.ds(base, ROWS_PER_W)], rows_v)
        pltpu.async_copy(rows_v, out_hbm.at[idx_v], sem).wait()

    @functools.partial(
        pl.kernel,
        mesh=mesh,
        out_type=jax.ShapeDtypeStruct((N, D), jnp.float32),
        scratch_types=scratch,
    )
    def sc_gather(y_hbm, pos_hbm, out_hbm, idx_v, rows_v, sem):
        wid = lax.axis_index("s") * NC + lax.axis_index("c")
        base = wid * ROWS_PER_W
        pltpu.sync_copy(pos_hbm.at[pl.ds(base, ROWS_PER_W)], idx_v)
        pltpu.async_copy(y_hbm.at[idx_v], rows_v, sem).wait()
        pltpu.sync_copy(rows_v, out_hbm.at[pl.ds(base, ROWS_PER_W)])

    return sc_scatter, sc_gather


def _ffn_body(tbl_ref, xs_ref, w1t_ref, b1_ref, w2_ref, b2_ref, y_ref):
    p = pl.program_id(0)
    for k in range(GC):
        r = tbl_ref[TP + GC * p + k]
        xs16 = xs_ref[pl.ds(k * B, B), :].astype(jnp.bfloat16)
        w1r = w1t_ref[r].astype(jnp.bfloat16)
        h = lax.dot_general(xs16, w1r, (((1,), (1,)), ((), ())),
                            preferred_element_type=jnp.float32)
        h = _gelu_exact(h + b1_ref[pl.ds(r, 1), :])
        y = lax.dot_general(h.astype(jnp.bfloat16), w2_ref[r],
                            (((1,), (0,)), ((), ())),
                            preferred_element_type=jnp.float32)
        y_ref[pl.ds(k * B, B), :] = y + b2_ref[pl.ds(r, 1), :]


def _grouped_ffn(xs_padded, tbl, w1t16, b1, w2_16, b2):
    grid_spec = pltpu.PrefetchScalarGridSpec(
        num_scalar_prefetch=1,
        grid=(TP,),
        in_specs=[
            pl.BlockSpec((GC * B, D), lambda p, tbl: (tbl[p], 0)),
            pl.BlockSpec((R, E, D), lambda p, tbl: (0, 0, 0)),
            pl.BlockSpec((R, E), lambda p, tbl: (0, 0)),
            pl.BlockSpec((R, E, D), lambda p, tbl: (0, 0, 0)),
            pl.BlockSpec((R, D), lambda p, tbl: (0, 0)),
        ],
        out_specs=pl.BlockSpec((GC * B, D), lambda p, tbl: (tbl[p], 0)),
    )
    return pl.pallas_call(
        _ffn_body,
        grid_spec=grid_spec,
        out_shape=jax.ShapeDtypeStruct((PN, D), jnp.float32),
        compiler_params=pltpu.CompilerParams(
            vmem_limit_bytes=110 * 1024 * 1024,
        ),
    )(tbl, xs_padded, w1t16, b1, w2_16, b2)


def kernel(x, rules, w1, b1, w2, b2):
    sc_scatter, sc_gather = _sc_kernels()
    w1t = jnp.swapaxes(w1, 1, 2)
    w2_16 = w2.astype(jnp.bfloat16)
    pos, tbl = _compute_routing(rules)
    xs_padded = sc_scatter(x, pos)
    y_padded = _grouped_ffn(xs_padded, tbl, w1t, b1, w2_16, b2)
    return sc_gather(y_padded, pos)


# speedup vs baseline: 1.2466x; 1.2466x over previous
import functools

import jax
import jax.numpy as jnp
from jax import lax
from jax.experimental import pallas as pl
from jax.experimental.pallas import tpu as pltpu
from jax.experimental.pallas import tpu_sc as plsc

N, D, E, R = 2048, 768, 64, 64
B = 64
SH = 6
GC = 16
PN = N + (R - 1) * B
PN = ((PN + GC * B - 1) // (GC * B)) * (GC * B)
T = PN // B
TP = T // GC
TBL = ((T + TP + 1 + 7) // 8) * 8

NC, NS = 2, 16
NW = NC * NS
ROWS_PER_W = N // NW

_SQRT_HALF = 0.7071067811865476


def _gelu_exact(v):
    return 0.5 * v * (1.0 + jax.lax.erf(v * _SQRT_HALF))


_RB = 1024
_RG = N // _RB


def _routing_body(rules_full_ref, rules_blk_ref, pos_ref, tbl_ref):
    pid = pl.program_id(0)
    rules_full = rules_full_ref[...]
    r_iota = lax.broadcasted_iota(jnp.int32, (R, N), 0)
    eqc = (r_iota == rules_full).astype(jnp.float32)
    counts_col = jnp.sum(eqc, axis=1, keepdims=True).astype(jnp.int32)
    padded_col = ((counts_col + (B - 1)) >> SH) << SH
    tri_r = (lax.broadcasted_iota(jnp.int32, (R, R), 1)
             < lax.broadcasted_iota(jnp.int32, (R, R), 0)).astype(jnp.float32)
    offsets_col = lax.dot_general(tri_r, padded_col.astype(jnp.float32),
                                  (((1,), (0,)), ((), ())),
                                  preferred_element_type=jnp.float32)

    @pl.when(pid == 0)
    def _chunks():
        nvalid = jnp.sum(padded_col) >> SH
        nvp = (nvalid + GC - 1) // GC
        p_iota = lax.broadcasted_iota(jnp.int32, (1, TP), 1)
        pair_tgt = jnp.minimum(p_iota, nvp)
        cb = (lax.broadcasted_iota(jnp.int32, (R, T), 1) * B).astype(jnp.float32)
        le = (offsets_col <= cb).astype(jnp.int32)
        rule = jnp.sum(le, axis=0, keepdims=True) - 1
        nv = jnp.full((1, TBL - T - TP), nvalid, jnp.int32)
        tbl_ref[...] = jnp.concatenate([pair_tgt, rule, nv], axis=1)

    rules_blk = rules_blk_ref[...]
    eqb = (lax.broadcasted_iota(jnp.int32, (R, _RB), 0)
           == rules_blk).astype(jnp.float32)
    before = (lax.broadcasted_iota(jnp.int32, (R, N), 1)
              < pid * _RB).astype(jnp.float32)
    pc_col = jnp.sum(eqc * before, axis=1, keepdims=True)
    tri_b = (lax.broadcasted_iota(jnp.int32, (_RB, _RB), 0)
             < lax.broadcasted_iota(jnp.int32, (_RB, _RB), 1)
             ).astype(jnp.float32)
    cblk = lax.dot_general(eqb, tri_b, (((1,), (0,)), ((), ())),
                           preferred_element_type=jnp.float32)
    rank = jnp.sum(eqb * (pc_col + cblk), axis=0, keepdims=True)
    off_tok = jnp.sum(eqb * offsets_col, axis=0, keepdims=True)
    pos_ref[...] = (off_tok + rank).astype(jnp.int32)


def _compute_routing(rules):
    rules_row = rules.reshape(1, N)
    pos, tbl = pl.pallas_call(
        _routing_body,
        grid=(_RG,),
        in_specs=[
            pl.BlockSpec((1, N), lambda i: (0, 0)),
            pl.BlockSpec((1, _RB), lambda i: (0, i)),
        ],
        out_specs=[
            pl.BlockSpec((1, _RB), lambda i: (0, i)),
            pl.BlockSpec((1, TBL), lambda i: (0, 0)),
        ],
        out_shape=[
            jax.ShapeDtypeStruct((1, N), jnp.int32),
            jax.ShapeDtypeStruct((1, TBL), jnp.int32),
        ],
    )(rules_row, rules_row)
    return pos.reshape(N), tbl.reshape(TBL)


@functools.lru_cache(maxsize=None)
def _sc_kernels():
    mesh = plsc.VectorSubcoreMesh(core_axis_name="c", subcore_axis_name="s")
    scratch = [
        pltpu.VMEM((ROWS_PER_W,), jnp.int32),
        pltpu.VMEM((ROWS_PER_W, D), jnp.float32),
        pltpu.SemaphoreType.DMA,
    ]

    @functools.partial(
        pl.kernel,
        mesh=mesh,
        out_type=jax.ShapeDtypeStruct((PN, D), jnp.float32),
        scratch_types=scratch,
    )
    def sc_scatter(x_hbm, pos_hbm, out_hbm, idx_v, rows_v, sem):
        wid = lax.axis_index("s") * NC + lax.axis_index("c")
        base = wid * ROWS_PER_W
        pltpu.sync_copy(pos_hbm.at[pl.ds(base, ROWS_PER_W)], idx_v)
        pltpu.sync_copy(x_hbm.at[pl.ds(base, ROWS_PER_W)], rows_v)
        pltpu.async_copy(rows_v, out_hbm.at[idx_v], sem).wait()

    @functools.partial(
        pl.kernel,
        mesh=mesh,
        out_type=jax.ShapeDtypeStruct((N, D), jnp.float32),
        scratch_types=scratch,
    )
    def sc_gather(y_hbm, pos_hbm, out_hbm, idx_v, rows_v, sem):
        wid = lax.axis_index("s") * NC + lax.axis_index("c")
        base = wid * ROWS_PER_W
        pltpu.sync_copy(pos_hbm.at[pl.ds(base, ROWS_PER_W)], idx_v)
        pltpu.async_copy(y_hbm.at[idx_v], rows_v, sem).wait()
        pltpu.sync_copy(rows_v, out_hbm.at[pl.ds(base, ROWS_PER_W)])

    return sc_scatter, sc_gather


def _ffn_body(tbl_ref, xs_ref, w1t_ref, b1_ref, w2_ref, b2_ref, y_ref):
    p = pl.program_id(0)
    for k in range(GC):
        r = tbl_ref[TP + GC * p + k]
        xs16 = xs_ref[pl.ds(k * B, B), :].astype(jnp.bfloat16)
        w1r = w1t_ref[r].astype(jnp.bfloat16)
        h = lax.dot_general(xs16, w1r, (((1,), (1,)), ((), ())),
                            preferred_element_type=jnp.float32)
        h = _gelu_exact(h + b1_ref[pl.ds(r, 1), :])
        y = lax.dot_general(h.astype(jnp.bfloat16), w2_ref[r],
                            (((1,), (0,)), ((), ())),
                            preferred_element_type=jnp.float32)
        y_ref[pl.ds(k * B, B), :] = y + b2_ref[pl.ds(r, 1), :]


def _grouped_ffn(xs_padded, tbl, w1t16, b1, w2_16, b2):
    grid_spec = pltpu.PrefetchScalarGridSpec(
        num_scalar_prefetch=1,
        grid=(TP,),
        in_specs=[
            pl.BlockSpec((GC * B, D), lambda p, tbl: (tbl[p], 0)),
            pl.BlockSpec((R, E, D), lambda p, tbl: (0, 0, 0)),
            pl.BlockSpec((R, E), lambda p, tbl: (0, 0)),
            pl.BlockSpec((R, E, D), lambda p, tbl: (0, 0, 0)),
            pl.BlockSpec((R, D), lambda p, tbl: (0, 0)),
        ],
        out_specs=pl.BlockSpec((GC * B, D), lambda p, tbl: (tbl[p], 0)),
    )
    return pl.pallas_call(
        _ffn_body,
        grid_spec=grid_spec,
        out_shape=jax.ShapeDtypeStruct((PN, D), jnp.float32),
        compiler_params=pltpu.CompilerParams(
            vmem_limit_bytes=110 * 1024 * 1024,
        ),
    )(tbl, xs_padded, w1t16, b1, w2_16, b2)


def kernel(x, rules, w1, b1, w2, b2):
    sc_scatter, sc_gather = _sc_kernels()
    w1t = jnp.swapaxes(w1, 1, 2)
    w2_16 = w2.astype(jnp.bfloat16)
    pos, tbl = _compute_routing(rules)
    xs_padded = sc_scatter(x, pos)
    y_padded = _grouped_ffn(xs_padded, tbl, w1t, b1, w2_16, b2)
    return sc_gather(y_padded, pos)
